# Initial kernel scaffold; baseline (speedup 1.0000x reference)
import jax
import jax.numpy as jnp
from jax.experimental import pallas as pl


def kernel(x, y):
    raise NotImplementedError("write your pallas kernel here")


# speedup vs baseline: 96.9991x; 96.9991x over previous
import jax
import jax.numpy as jnp
from jax.experimental import pallas as pl
from jax.experimental.pallas import tpu as pltpu

_EPS = 1e-12
_C = 2048
_NEG = -jnp.inf


def _topk_kernel(y_ref, x_ref, o_ref, m1, m2, m3, idx, acc, *, rows, cols, nc):
    j = pl.program_id(1)
    r = rows

    @pl.when(j == 0)
    def _init():
        m1[...] = jnp.full((r, 128), _NEG, jnp.float32)
        m2[...] = jnp.full((r, 128), _NEG, jnp.float32)
        m3[...] = jnp.full((r, 128), _NEG, jnp.float32)
        idx[...] = jnp.zeros((r, 128), jnp.int32)
        acc[...] = jnp.zeros((r, 128), jnp.float32)

    yb = y_ref[0, 0, :][:, None]
    lane = jax.lax.broadcasted_iota(jnp.int32, (r, 128), 1)
    base = j * _C
    for s in range(_C // 128):
        v = x_ref[:, s * 128:(s + 1) * 128]
        cidx = lane + (base + s * 128)
        v = jnp.where(cidx < cols, v, _NEG)
        ge = v >= m1[...]
        idx[...] = jnp.where(ge, cidx, idx[...])
        om1 = m1[...]
        om2 = m2[...]
        m1[...] = jnp.maximum(om1, v)
        m2[...] = jnp.minimum(om1, jnp.maximum(om2, v))
        m3[...] = jnp.minimum(om2, jnp.maximum(m3[...], v))
        acc[...] += jnp.where(cidx == yb, v, 0.0)

    @pl.when(j == nc - 1)
    def _finish():
        lanes = jax.lax.broadcasted_iota(jnp.int32, (r, 128), 1)
        a1 = m1[...]
        big1 = jnp.max(a1, axis=1, keepdims=True)
        idxmax = jnp.max(jnp.where(a1 == big1, idx[...], -1), axis=1,
                         keepdims=True)
        l1 = jnp.max(jnp.where(a1 == big1, lanes, -1), axis=1, keepdims=True)
        a2 = jnp.where(lanes == l1, m2[...], a1)
        big2 = jnp.max(a2, axis=1, keepdims=True)
        l2 = jnp.max(jnp.where(a2 == big2, lanes, -1), axis=1, keepdims=True)
        a3 = jnp.where(lanes == l2, jnp.where(l1 == l2, m3[...], m2[...]), a2)
        big3 = jnp.max(a3, axis=1, keepdims=True)
        xy = jnp.sum(acc[...], axis=1, keepdims=True)
        ind = idxmax == yb
        num = xy - jnp.where(ind, big2, big1)
        den = big1 - big3 + _EPS
        res = -num / den
        o_ref[0, 0, :] = res[:, 0]


def kernel(x, y):
    rows, cols = x.shape
    r = 128 if rows % 128 == 0 else rows
    nr = rows // r
    nc = pl.cdiv(cols, _C)
    y32 = y.astype(jnp.int32).reshape(nr, 1, r)

    import functools
    body = functools.partial(_topk_kernel, rows=r, cols=cols, nc=nc)
    out = pl.pallas_call(
        body,
        grid=(nr, nc),
        in_specs=[
            pl.BlockSpec((1, 1, r), lambda i, j: (i, 0, 0)),
            pl.BlockSpec((r, _C), lambda i, j: (i, j)),
        ],
        out_specs=pl.BlockSpec((1, 1, r), lambda i, j: (i, 0, 0)),
        out_shape=jax.ShapeDtypeStruct((nr, 1, r), jnp.float32),
        scratch_shapes=[
            pltpu.VMEM((r, 128), jnp.float32),
            pltpu.VMEM((r, 128), jnp.float32),
            pltpu.VMEM((r, 128), jnp.float32),
            pltpu.VMEM((r, 128), jnp.int32),
            pltpu.VMEM((r, 128), jnp.float32),
        ],
        compiler_params=pltpu.CompilerParams(
            dimension_semantics=("arbitrary", "arbitrary")),
    )(y32, x)
    return out.reshape(rows)
